# Initial kernel scaffold; baseline (speedup 1.0000x reference)
import jax
import jax.numpy as jnp
from jax.experimental import pallas as pl


def kernel(outputs, complementary_labels):
    raise NotImplementedError("write your pallas kernel here")


# speedup vs baseline: 2.6760x; 2.6760x over previous
import functools

import jax
import jax.numpy as jnp
from jax import lax
from jax.experimental import pallas as pl
from jax.experimental.pallas import tpu as pltpu
from jax.experimental.pallas import tpu_sc as plsc

_B = 4096
_C = 1000
_K = 10

_NC = 2
_NS = 16
_L = 16
_NW = _NC * _NS
_RPW = _B // _NW
_NCH = _RPW // _L


def _sc_body(out_flat_hbm, lab_flat_hbm, part_hbm, lab_v, idx_v, val_v,
             acc_v, sem):
    cid = lax.axis_index("c")
    sid = lax.axis_index("s")
    wid = sid * _NC + cid
    base_row = wid * _RPW

    pltpu.sync_copy(lab_flat_hbm.at[pl.ds(base_row * _K, _RPW * _K)], lab_v)

    lane = lax.iota(jnp.int32, _L)
    for c in range(_NCH):
        r = c * _L + lane
        row_term = (base_row + r) * _C
        for j in range(_K):
            labs = plsc.load_gather(lab_v, [r * _K + j])
            idx_v[j, pl.ds(c * _L, _L)] = row_term + labs

    copies = [
        pltpu.async_copy(out_flat_hbm.at[idx_v.at[j]], val_v.at[j], sem)
        for j in range(_K)
    ]
    for cp in copies:
        cp.wait()

    acc = jnp.zeros((_L,), jnp.float32)
    for c in range(_NCH):
        sl = pl.ds(c * _L, _L)
        idxs = [idx_v[j, sl] for j in range(_K)]
        vals = [val_v[j, sl] for j in range(_K)]
        acc = acc + jnp.exp(vals[0])
        for j in range(1, _K):
            first = idxs[j] != idxs[0]
            for k in range(1, j):
                first = jnp.logical_and(first, idxs[j] != idxs[k])
            acc = acc + jnp.where(first, jnp.exp(vals[j]), 0.0)

    acc_v[...] = acc
    pltpu.sync_copy(acc_v, part_hbm.at[wid])


_sc_loss = functools.partial(
    pl.kernel,
    mesh=plsc.VectorSubcoreMesh(core_axis_name="c", subcore_axis_name="s",
                                num_cores=_NC, num_subcores=_NS),
    compiler_params=pltpu.CompilerParams(needs_layout_passes=False),
    out_type=jax.ShapeDtypeStruct((_NW, _L), jnp.float32),
    scratch_types=[
        pltpu.VMEM((_RPW * _K,), jnp.int32),
        pltpu.VMEM((_K, _RPW), jnp.int32),
        pltpu.VMEM((_K, _RPW), jnp.float32),
        pltpu.VMEM((_L,), jnp.float32),
        pltpu.SemaphoreType.DMA,
    ],
)(_sc_body)


def kernel(outputs, complementary_labels):
    partials = _sc_loss(outputs.reshape(-1), complementary_labels.reshape(-1))
    return partials.sum() / outputs.shape[0]


# speedup vs baseline: 3.4558x; 1.2914x over previous
import functools

import jax
import jax.numpy as jnp
from jax import lax
from jax.experimental import pallas as pl
from jax.experimental.pallas import tpu as pltpu
from jax.experimental.pallas import tpu_sc as plsc

_B = 4096
_C = 1000
_K = 10

_NC = 2
_NS = 16
_L = 16
_NW = _NC * _NS
_RPW = _B // _NW
_NCH = _RPW // _L


def _sc_body(out_hbm, lab_flat_hbm, part_hbm, lab_v, row_v, acc_v, sem0,
             sem1):
    cid = lax.axis_index("c")
    sid = lax.axis_index("s")
    wid = sid * _NC + cid
    base_row = wid * _RPW
    sems = (sem0, sem1)

    def start(c):
        return pltpu.async_copy(
            out_hbm.at[pl.ds(base_row + c * _L, _L), :],
            row_v.at[c % 2], sems[c % 2])

    copies = {0: start(0)}
    pltpu.sync_copy(lab_flat_hbm.at[pl.ds(base_row * _K, _RPW * _K)], lab_v)

    lane = lax.iota(jnp.int32, _L)
    acc = jnp.zeros((_L,), jnp.float32)
    for c in range(_NCH):
        if c + 1 < _NCH:
            copies[c + 1] = start(c + 1)
        copies[c].wait()
        rows = row_v.at[c % 2]
        labs = [
            plsc.load_gather(lab_v, [(c * _L + lane) * _K + j])
            for j in range(_K)
        ]
        vals = [plsc.load_gather(rows, [lane, labs[j]]) for j in range(_K)]
        acc = acc + jnp.exp(vals[0])
        for j in range(1, _K):
            first = labs[j] != labs[0]
            for k in range(1, j):
                first = jnp.logical_and(first, labs[j] != labs[k])
            acc = acc + jnp.where(first, jnp.exp(vals[j]), 0.0)

    acc_v[...] = acc
    pltpu.sync_copy(acc_v, part_hbm.at[wid])


_sc_loss = functools.partial(
    pl.kernel,
    mesh=plsc.VectorSubcoreMesh(core_axis_name="c", subcore_axis_name="s",
                                num_cores=_NC, num_subcores=_NS),
    compiler_params=pltpu.CompilerParams(needs_layout_passes=False),
    out_type=jax.ShapeDtypeStruct((_NW, _L), jnp.float32),
    scratch_types=[
        pltpu.VMEM((_RPW * _K,), jnp.int32),
        pltpu.VMEM((2, _L, _C), jnp.float32),
        pltpu.VMEM((_L,), jnp.float32),
        pltpu.SemaphoreType.DMA,
        pltpu.SemaphoreType.DMA,
    ],
)(_sc_body)


def kernel(outputs, complementary_labels):
    partials = _sc_loss(outputs, complementary_labels.reshape(-1))
    return partials.sum() / outputs.shape[0]


# speedup vs baseline: 5.5743x; 1.6130x over previous
import functools

import jax
import jax.numpy as jnp
from jax import lax
from jax.experimental import pallas as pl
from jax.experimental.pallas import tpu as pltpu
from jax.experimental.pallas import tpu_sc as plsc

_B = 4096
_C = 1000
_K = 10

_NC = 2
_NS = 16
_L = 16
_NW = _NC * _NS
_RPW = _B // _NW
_NCH = _RPW // _L


def _sc_body(outT_hbm, lab_flat_hbm, part_hbm, lab_v, row_v, acc_v, sem):
    cid = lax.axis_index("c")
    sid = lax.axis_index("s")
    wid = sid * _NC + cid
    base_row = wid * _RPW

    cp = pltpu.async_copy(outT_hbm.at[:, pl.ds(base_row, _RPW)], row_v, sem)
    pltpu.sync_copy(lab_flat_hbm.at[pl.ds(base_row * _K, _RPW * _K)], lab_v)
    cp.wait()

    lane = lax.iota(jnp.int32, _L)
    acc = jnp.zeros((_L,), jnp.float32)
    for c in range(_NCH):
        col = c * _L + lane
        labs = [
            plsc.load_gather(lab_v, [col * _K + j]) for j in range(_K)
        ]
        vals = [plsc.load_gather(row_v, [labs[j], col]) for j in range(_K)]
        acc = acc + jnp.exp(vals[0])
        for j in range(1, _K):
            first = labs[j] != labs[0]
            for k in range(1, j):
                first = jnp.logical_and(first, labs[j] != labs[k])
            acc = acc + jnp.where(first, jnp.exp(vals[j]), 0.0)

    acc_v[...] = acc
    pltpu.sync_copy(acc_v, part_hbm.at[wid])


_sc_loss = functools.partial(
    pl.kernel,
    mesh=plsc.VectorSubcoreMesh(core_axis_name="c", subcore_axis_name="s",
                                num_cores=_NC, num_subcores=_NS),
    compiler_params=pltpu.CompilerParams(needs_layout_passes=False,
                                         use_tc_tiling_on_sc=True),
    out_type=jax.ShapeDtypeStruct((_NW, _L), jnp.float32),
    scratch_types=[
        pltpu.VMEM((_RPW * _K,), jnp.int32),
        pltpu.VMEM((_C, _RPW), jnp.float32),
        pltpu.VMEM((_L,), jnp.float32),
        pltpu.SemaphoreType.DMA,
    ],
)(_sc_body)


def kernel(outputs, complementary_labels):
    partials = _sc_loss(outputs.T, complementary_labels.reshape(-1))
    return partials.sum() / outputs.shape[0]


# speedup vs baseline: 5.7235x; 1.0268x over previous
import functools

import jax
import jax.numpy as jnp
from jax import lax
from jax.experimental import pallas as pl
from jax.experimental.pallas import tpu as pltpu
from jax.experimental.pallas import tpu_sc as plsc

_B = 4096
_C = 1000
_K = 10

_NC = 2
_NS = 16
_L = 16
_NW = _NC * _NS
_RPW = _B // _NW
_NCH = _RPW // _L


def _sc_body(outT_hbm, labT_hbm, part_hbm, lab_v, row_v, acc_v, sem):
    cid = lax.axis_index("c")
    sid = lax.axis_index("s")
    wid = sid * _NC + cid
    base_row = wid * _RPW

    cp = pltpu.async_copy(outT_hbm.at[:, pl.ds(base_row, _RPW)], row_v, sem)
    pltpu.sync_copy(labT_hbm.at[:, pl.ds(base_row, _RPW)], lab_v)
    cp.wait()

    lane = lax.iota(jnp.int32, _L)
    acc = jnp.zeros((_L,), jnp.float32)
    for c in range(_NCH):
        col = c * _L + lane
        labs = [lab_v[j, pl.ds(c * _L, _L)] for j in range(_K)]
        vals = [plsc.load_gather(row_v, [labs[j], col]) for j in range(_K)]
        acc = acc + jnp.exp(vals[0])
        for j in range(1, _K):
            first = labs[j] != labs[0]
            for k in range(1, j):
                first = jnp.logical_and(first, labs[j] != labs[k])
            acc = acc + jnp.where(first, jnp.exp(vals[j]), 0.0)

    acc_v[...] = acc
    pltpu.sync_copy(acc_v, part_hbm.at[wid])


_sc_loss = functools.partial(
    pl.kernel,
    mesh=plsc.VectorSubcoreMesh(core_axis_name="c", subcore_axis_name="s",
                                num_cores=_NC, num_subcores=_NS),
    compiler_params=pltpu.CompilerParams(needs_layout_passes=False,
                                         use_tc_tiling_on_sc=True,
                                         skip_device_barrier=True),
    out_type=jax.ShapeDtypeStruct((_NW, _L), jnp.float32),
    scratch_types=[
        pltpu.VMEM((_K, _RPW), jnp.int32),
        pltpu.VMEM((_C, _RPW), jnp.float32),
        pltpu.VMEM((_L,), jnp.float32),
        pltpu.SemaphoreType.DMA,
    ],
)(_sc_body)


def kernel(outputs, complementary_labels):
    partials = _sc_loss(outputs.T, complementary_labels.T)
    return partials.sum() / outputs.shape[0]
